# Initial kernel scaffold; baseline (speedup 1.0000x reference)
import jax
import jax.numpy as jnp
from jax.experimental import pallas as pl


def kernel(x, edge_index, W1, b1, W2, b2):
    raise NotImplementedError("write your pallas kernel here")


# speedup vs baseline: 4.6622x; 4.6622x over previous
import functools

import jax
import jax.numpy as jnp
from jax import lax
from jax.experimental import pallas as pl
from jax.experimental.pallas import tpu as pltpu
from jax.experimental.pallas import tpu_sc as plsc

NC, NS, L = 2, 16, 16
NW = NC * NS
ECH = 2000
R = 80


def _sc_mesh():
    return plsc.VectorSubcoreMesh(
        core_axis_name="c", subcore_axis_name="s",
        num_cores=NC, num_subcores=NS)


def _wid():
    return lax.axis_index("s") * NC + lax.axis_index("c")


def _degree_partials(cols, n_pad):
    e = cols.shape[0]
    ew = e // NW
    n_chunks = ew // ECH

    def body(cols_hbm, out_hbm, hist_v, cbuf):
        wid = _wid()
        zeros = jnp.zeros((L,), jnp.float32)
        ones = jnp.ones((L,), jnp.float32)

        def zero_body(i, _):
            hist_v[pl.ds(i * L, L)] = zeros
            return 0
        lax.fori_loop(0, n_pad // L, zero_body, 0)

        base = wid * ew
        for ch in range(n_chunks):
            pltpu.sync_copy(cols_hbm.at[pl.ds(base + ch * ECH, ECH)], cbuf)

            def scan_body(v, _):
                c = cbuf[pl.ds(v * L, L)]
                plsc.addupdate_scatter(hist_v, [c], ones)
                return 0
            lax.fori_loop(0, ECH // L, scan_body, 0)

        pltpu.sync_copy(hist_v, out_hbm.at[wid])

    f = pl.kernel(
        body,
        out_type=jax.ShapeDtypeStruct((NW, n_pad), jnp.float32),
        mesh=_sc_mesh(),
        compiler_params=pltpu.CompilerParams(needs_layout_passes=False),
        scratch_types=[
            pltpu.VMEM((n_pad,), jnp.float32),
            pltpu.VMEM((ECH,), jnp.int32),
        ],
    )
    return f(cols)


def _tc_dis(parts):
    n_pad = parts.shape[1]

    def body(p_ref, o_ref):
        s = jnp.sum(p_ref[...], axis=0) + 1.0
        o_ref[...] = lax.rsqrt(s)[None, :]

    return pl.pallas_call(
        body,
        out_shape=jax.ShapeDtypeStruct((1, n_pad), jnp.float32),
    )(parts)


def _tc_scaled_mm(x_pad, w, dis, t=1024):
    b_sz, n_pad, f = x_pad.shape
    grid = (n_pad // t, b_sz)

    def body(x_ref, w_ref, d_ref, o_ref):
        y = jnp.dot(x_ref[0], w_ref[...], preferred_element_type=jnp.float32)
        o_ref[...] = y * d_ref[0][:, None]

    return pl.pallas_call(
        body,
        grid=grid,
        in_specs=[
            pl.BlockSpec((1, t, f), lambda i, b: (b, i, 0)),
            pl.BlockSpec((f, f), lambda i, b: (0, 0)),
            pl.BlockSpec((1, t), lambda i, b: (0, i)),
        ],
        out_specs=pl.BlockSpec((t, f), lambda i, b: (i, b)),
        out_shape=jax.ShapeDtypeStruct((n_pad, b_sz * f), jnp.float32),
    )(x_pad, w, dis)


def _tc_mid(acc, dis, b1, w2, t=1024):
    n_pad, bf = acc.shape
    f = w2.shape[0]
    grid = (n_pad // t, bf // f)

    def body(a_ref, d_ref, b_ref, w_ref, o_ref):
        d = d_ref[0][:, None]
        h = jnp.maximum(a_ref[...] * d + b_ref[...], 0.0)
        o_ref[...] = jnp.dot(h, w_ref[...],
                             preferred_element_type=jnp.float32) * d

    return pl.pallas_call(
        body,
        grid=grid,
        in_specs=[
            pl.BlockSpec((t, f), lambda i, b: (i, b)),
            pl.BlockSpec((1, t), lambda i, b: (0, i)),
            pl.BlockSpec((1, f), lambda i, b: (0, 0)),
            pl.BlockSpec((f, f), lambda i, b: (0, 0)),
        ],
        out_specs=pl.BlockSpec((t, f), lambda i, b: (i, b)),
        out_shape=jax.ShapeDtypeStruct((n_pad, bf), jnp.float32),
    )(acc, dis, b1, w2)


def _tc_final(acc, dis, b2, t=1024):
    n_pad, bf = acc.shape
    f = b2.shape[1]
    b_sz = bf // f
    grid = (n_pad // t, b_sz)

    def body(a_ref, d_ref, b_ref, o_ref):
        o_ref[0] = a_ref[...] * d_ref[0][:, None] + b_ref[...]

    return pl.pallas_call(
        body,
        grid=grid,
        in_specs=[
            pl.BlockSpec((t, f), lambda i, b: (i, b)),
            pl.BlockSpec((1, t), lambda i, b: (0, i)),
            pl.BlockSpec((1, f), lambda i, b: (0, 0)),
        ],
        out_specs=pl.BlockSpec((1, t, f), lambda i, b: (b, i, 0)),
        out_shape=jax.ShapeDtypeStruct((b_sz, n_pad, f), jnp.float32),
    )(acc, dis, b2)


def _sc_spmm(y, rows, cols):
    n_pad, bf = y.shape
    e = rows.shape[0]
    n_passes = n_pad // (NW * R)
    assert n_pad % (NW * R) == 0 and e % ECH == 0

    def body(y_hbm, rows_hbm, cols_hbm, out_hbm,
             acc, staged, rbuf, cbuf, rowbuf, lcolbuf, gsem):
        wid = _wid()
        lane = lax.iota(jnp.int32, L)
        pad_l = jnp.full((L,), R, jnp.int32)
        zero_l = jnp.zeros((L,), jnp.int32)

        for p in range(n_passes):
            lo = (p * NW + wid) * R
            pltpu.sync_copy(y_hbm.at[pl.ds(lo, R)], acc.at[pl.ds(0, R)])

            def chunk_body(ec, _):
                pltpu.sync_copy(rows_hbm.at[pl.ds(ec * ECH, ECH)], rbuf)
                pltpu.sync_copy(cols_hbm.at[pl.ds(ec * ECH, ECH)], cbuf)

                def scan_body(v, cnt):
                    c = cbuf[pl.ds(v * L, L)]
                    r = rbuf[pl.ds(v * L, L)]
                    m = (c >= lo) & (c < lo + R)
                    plsc.store_compressed(rowbuf.at[pl.ds(cnt, L)], r, mask=m)
                    plsc.store_compressed(lcolbuf.at[pl.ds(cnt, L)],
                                          c - lo, mask=m)
                    return cnt + jnp.sum(m.astype(jnp.int32))

                cnt = lax.fori_loop(0, ECH // L, scan_body,
                                    jnp.int32(0))
                rowbuf[pl.ds(cnt, L)] = zero_l
                lcolbuf[pl.ds(cnt, L)] = pad_l
                ng = (cnt + (L - 1)) // L

                def drain_body(g, _):
                    rvec = rowbuf[pl.ds(g * L, L)]
                    lvec = lcolbuf[pl.ds(g * L, L)]
                    desc = pltpu.make_async_copy(
                        y_hbm.at[rvec], staged, gsem)
                    desc.start()
                    desc.wait()

                    def edge_body(k, _):
                        kf = jnp.broadcast_to(k, (L,)).astype(jnp.int32)
                        lc = jnp.sum(jnp.where(lane == k, lvec, 0))
                        lcf = jnp.broadcast_to(lc, (L,))
                        for j in range(bf // L):
                            cv = lane + (j * L)
                            xv = plsc.load_gather(staged, [kf, cv])
                            plsc.addupdate_scatter(acc, [lcf, cv], xv)
                        return 0

                    lax.fori_loop(0, L, edge_body, 0)
                    return 0

                lax.fori_loop(0, ng, drain_body, 0)
                return 0

            lax.fori_loop(0, e // ECH, chunk_body, 0)
            pltpu.sync_copy(acc.at[pl.ds(0, R)], out_hbm.at[pl.ds(lo, R)])

    f = pl.kernel(
        body,
        out_type=jax.ShapeDtypeStruct((n_pad, bf), jnp.float32),
        mesh=_sc_mesh(),
        compiler_params=pltpu.CompilerParams(needs_layout_passes=False),
        scratch_types=[
            pltpu.VMEM((R + 1, bf), jnp.float32),
            pltpu.VMEM((L, bf), jnp.float32),
            pltpu.VMEM((ECH,), jnp.int32),
            pltpu.VMEM((ECH,), jnp.int32),
            pltpu.VMEM((ECH + L,), jnp.int32),
            pltpu.VMEM((ECH + L,), jnp.int32),
            pltpu.SemaphoreType.DMA,
        ],
    )
    return f(y, rows, cols)


def kernel(x, edge_index, W1, b1, W2, b2):
    b_sz, n, f = x.shape
    n_pad = NW * R * -(-n // (NW * R))

    rows = edge_index[0].astype(jnp.int32)
    cols = edge_index[1].astype(jnp.int32)
    e = rows.shape[0]
    e_pad = NW * ECH * -(-e // (NW * ECH))
    if e_pad != e:
        rows = jnp.concatenate(
            [rows, jnp.zeros((e_pad - e,), jnp.int32)])
        cols = jnp.concatenate(
            [cols, jnp.full((e_pad - e,), n, jnp.int32)])

    x_pad = jnp.pad(x, ((0, 0), (0, n_pad - n), (0, 0)))
    b1r = b1.reshape(1, -1)
    b2r = b2.reshape(1, -1)

    parts = _degree_partials(cols, n_pad)
    dis = _tc_dis(parts)

    y1 = _tc_scaled_mm(x_pad, W1, dis)
    acc1 = _sc_spmm(y1, rows, cols)
    y2 = _tc_mid(acc1, dis, b1r, W2)
    acc2 = _sc_spmm(y2, rows, cols)
    out = _tc_final(acc2, dis, b2r)
    return out[:, :n, :]
